# baseline (speedup 1.0000x reference)
import jax
import jax.numpy as jnp
from jax import lax
from jax.experimental import pallas as pl
from jax.experimental.pallas import tpu as pltpu
from jax.experimental.pallas import tpu_sc as plsc

N = 10000
D = 128
H = 512
E2 = 640000
DUMMY = N
AGG_ROWS = 10112
ROWS_PER_TILE = AGG_ROWS // 16
NW = 32
KCH = 160
CH = 128
E_CAP = NW * KCH * CH
RB = 1000
NRB = N // RB


GCH = 16
NGRP = KCH // GCH


def _spmm_body(src_hbm, dst_hbm, x_hbm, zrows_hbm, out_hbm,
               sidx, didx, buf0, buf1, sg0, sg1, ss0, ss1, agg_sh):
    c = lax.axis_index("c")
    s = lax.axis_index("s")
    wid = s * 2 + c
    tid = s
    pltpu.sync_copy(zrows_hbm, agg_sh.at[pl.ds(tid * ROWS_PER_TILE, ROWS_PER_TILE)])
    plsc.subcore_barrier()

    def g_start(ch, buf, sem):
        pltpu.make_async_copy(x_hbm.at[sidx.at[ch]], buf, sem).start()

    def g_wait(buf, sem):
        pltpu.make_async_copy(x_hbm.at[sidx.at[0]], buf, sem).wait()

    def s_start(ch, buf, sem):
        pltpu.async_copy(buf, agg_sh.at[didx.at[ch]], sem, add=True)

    def s_wait(buf, sem):
        pltpu.make_async_copy(buf, agg_sh.at[didx.at[0]], sem).wait()

    def group(grp, carry):
        pltpu.sync_copy(src_hbm.at[wid, pl.ds(grp * GCH, GCH)], sidx)
        pltpu.sync_copy(dst_hbm.at[wid, pl.ds(grp * GCH, GCH)], didx)
        g_start(0, buf0, sg0)
        g_start(1, buf1, sg1)

        def body(i, carry2):
            cc = i * 2
            g_wait(buf0, sg0)
            s_start(cc, buf0, ss0)
            g_wait(buf1, sg1)
            s_start(cc + 1, buf1, ss1)
            s_wait(buf0, ss0)
            g_start(cc + 2, buf0, sg0)
            s_wait(buf1, ss1)
            g_start(cc + 3, buf1, sg1)
            return carry2

        lax.fori_loop(0, GCH // 2 - 1, body, 0)
        g_wait(buf0, sg0)
        s_start(GCH - 2, buf0, ss0)
        g_wait(buf1, sg1)
        s_start(GCH - 1, buf1, ss1)
        s_wait(buf0, ss0)
        s_wait(buf1, ss1)
        return carry

    lax.fori_loop(0, NGRP, group, 0)
    plsc.subcore_barrier()
    pltpu.sync_copy(agg_sh.at[pl.ds(tid * ROWS_PER_TILE, ROWS_PER_TILE)],
                    out_hbm.at[c, pl.ds(tid * ROWS_PER_TILE, ROWS_PER_TILE)])


import functools


@functools.cache
def _spmm_kernel():
    return pl.kernel(
        _spmm_body,
        out_type=jax.ShapeDtypeStruct((2, AGG_ROWS, D), jnp.float32),
        mesh=plsc.VectorSubcoreMesh(core_axis_name="c", subcore_axis_name="s",
                                    num_cores=2, num_subcores=16),
        scratch_types=[
            pltpu.VMEM((GCH, CH), jnp.int32),
            pltpu.VMEM((GCH, CH), jnp.int32),
            pltpu.VMEM((CH, D), jnp.float32),
            pltpu.VMEM((CH, D), jnp.float32),
            pltpu.SemaphoreType.DMA,
            pltpu.SemaphoreType.DMA,
            pltpu.SemaphoreType.DMA,
            pltpu.SemaphoreType.DMA,
            pltpu.VMEM_SHARED((AGG_ROWS, D), jnp.float32),
        ],
    )


def _spmm(src3, dst3, h, zrows):
    return _spmm_kernel()(src3, dst3, h, zrows)


def _layer_body(x_ref, a_ref, w1_ref, g1_ref, b1_ref, w2_ref,
                og_ref, ob_ref, rmout_ref, o_ref, t_scr, y_scr, s1_scr, s2_scr):
    p = pl.program_id(0)
    b = pl.program_id(1)

    @pl.when(p == 0)
    def _():
        h = x_ref[...] + a_ref[0] + a_ref[1]
        t = jnp.dot(h, w1_ref[...], preferred_element_type=jnp.float32)
        t_scr[pl.ds(b * RB, RB), :] = t

        @pl.when(b == 0)
        def _():
            s1_scr[...] = jnp.zeros_like(s1_scr)

        s1_scr[0:1, :] += jnp.sum(t, axis=0, keepdims=True)
        s1_scr[1:2, :] += jnp.sum(t * t, axis=0, keepdims=True)

    @pl.when(p == 1)
    def _():
        t = t_scr[pl.ds(b * RB, RB), :]
        m = s1_scr[0:1, :] * (1.0 / N)
        v = s1_scr[1:2, :] * (1.0 / N) - m * m
        tn = g1_ref[...] * (t - m) / jnp.sqrt(v + 1e-5) + b1_ref[...]
        tr = jnp.maximum(tn, 0.0)
        y = jnp.dot(tr, w2_ref[...], preferred_element_type=jnp.float32)
        y_scr[pl.ds(b * RB, RB), :] = y

        @pl.when(b == 0)
        def _():
            s2_scr[...] = jnp.zeros_like(s2_scr)

        s2_scr[0:1, :] += jnp.sum(y, axis=0, keepdims=True)
        s2_scr[1:2, :] += jnp.sum(y * y, axis=0, keepdims=True)

    @pl.when(p == 2)
    def _():
        y = y_scr[pl.ds(b * RB, RB), :]
        m = s2_scr[0:1, :] * (1.0 / N)
        v = s2_scr[1:2, :] * (1.0 / N) - m * m
        yn = og_ref[...] * (y - m) / jnp.sqrt(v + 1e-5) + ob_ref[...]
        o_ref[...] = jnp.maximum(yn, 0.0) * rmout_ref[...]


def _layer(h, partials, w1, g1, b1, w2, og, ob, rmout):
    return pl.pallas_call(
        _layer_body,
        grid=(3, NRB),
        in_specs=[
            pl.BlockSpec((RB, D), lambda p, b: ((p == 0) * b, 0)),
            pl.BlockSpec((2, RB, D), lambda p, b: (0, (p == 0) * b, 0)),
            pl.BlockSpec((D, H), lambda p, b: (0, 0)),
            pl.BlockSpec((1, H), lambda p, b: (0, 0)),
            pl.BlockSpec((1, H), lambda p, b: (0, 0)),
            pl.BlockSpec((H, D), lambda p, b: (0, 0)),
            pl.BlockSpec((1, D), lambda p, b: (0, 0)),
            pl.BlockSpec((1, D), lambda p, b: (0, 0)),
            pl.BlockSpec((RB, 1), lambda p, b: ((p == 2) * b, 0)),
        ],
        out_specs=pl.BlockSpec((RB, D), lambda p, b: ((p == 2) * b, 0)),
        out_shape=jax.ShapeDtypeStruct((N, D), jnp.float32),
        scratch_shapes=[
            pltpu.VMEM((N, H), jnp.float32),
            pltpu.VMEM((N, D), jnp.float32),
            pltpu.VMEM((2, H), jnp.float32),
            pltpu.VMEM((2, D), jnp.float32),
        ],
    )(h, partials, w1, g1, b1, w2, og, ob, rmout)


def _rowmask_body(x_ref, rm_ref, o_ref):
    o_ref[...] = x_ref[...] * rm_ref[...]


def _rowmask(x, rm):
    return pl.pallas_call(
        _rowmask_body,
        grid=(NRB,),
        in_specs=[
            pl.BlockSpec((RB, D), lambda b: (b, 0)),
            pl.BlockSpec((RB, 1), lambda b: (b, 0)),
        ],
        out_specs=pl.BlockSpec((RB, D), lambda b: (b, 0)),
        out_shape=jax.ShapeDtypeStruct((N, D), jnp.float32),
    )(x, rm)


def _loss_body(x_ref, r1_ref, r2_ref, wm1_ref, wm2_ref, o_ref):
    def nrm(a):
        nn = jnp.sqrt(jnp.sum(a * a, axis=1, keepdims=True))
        return a / jnp.maximum(nn, 1e-12)

    xn = nrm(x_ref[...])
    r1n = nrm(r1_ref[...])
    r2n = nrm(r2_ref[...])
    c1 = jnp.sum(r1n * xn, axis=1, keepdims=True)
    c2 = jnp.sum(r2n * xn, axis=1, keepdims=True)
    ccl = jnp.sum(r2n * r1n, axis=1, keepdims=True)
    l1 = jnp.sum(wm1_ref[...] * (1.0 - c1)) / (N // 2)
    l2 = jnp.sum(wm2_ref[...] * (1.0 - c2)) / (N // 2)
    cl = jnp.sum(1.0 - ccl) / N
    o_ref[...] = jnp.reshape(l1 + l2 + 0.1 * cl, (1, 1))


def _loss(x, r1, r2, wm1, wm2):
    return pl.pallas_call(
        _loss_body,
        out_shape=jax.ShapeDtypeStruct((1, 1), jnp.float32),
    )(x, r1, r2, wm1, wm2)


def _tf2x32(k0, k1, x0, x1):
    ks = (k0, k1, k0 ^ k1 ^ jnp.uint32(0x1BD11BDA))
    rotations = ((13, 15, 26, 6), (17, 29, 16, 24))
    x0 = x0 + ks[0]
    x1 = x1 + ks[1]
    for i in range(5):
        for r in rotations[i % 2]:
            x0 = x0 + x1
            x1 = (x1 << r) | (x1 >> (32 - r))
            x1 = x0 ^ x1
        x0 = x0 + ks[(i + 1) % 3]
        x1 = x1 + ks[(i + 2) % 3] + jnp.uint32(i + 1)
    return x0, x1


def _unif_prefix(key, count, size):
    kd = jax.random.key_data(key)
    c = count.astype(jnp.uint32)
    half = (c + jnp.uint32(1)) // jnp.uint32(2)
    odd = c & jnp.uint32(1)
    j = jnp.arange(size, dtype=jnp.uint32)
    second = j >= half
    a0 = jnp.where(second, j - half, j)
    pad = (odd == jnp.uint32(1)) & (j == half - jnp.uint32(1))
    a1 = jnp.where(second, j, jnp.where(pad, jnp.uint32(0), j + half))
    o0, o1 = _tf2x32(kd[0], kd[1], a0, a1)
    bits = jnp.where(second, o1, o0)
    f = jax.lax.bitcast_convert_type((bits >> 9) | jnp.uint32(0x3F800000), jnp.float32)
    return f - jnp.float32(1.0)


def _keep_bits(key, valid, p=0.2):
    size = valid.shape[0]
    if jax.config.jax_threefry_partitionable:
        return jax.random.bernoulli(key, 1.0 - p, (size,))
    u = _unif_prefix(key, valid.sum(), size)
    return u < jnp.float32(1.0 - p)


def kernel(x, edge_index, batch, e0_w1, e0_g1, e0_b1, e0_w2, e0_og, e0_ob,
           e1_w1, e1_g1, e1_b1, e1_w2, e1_og, e1_ob,
           d0_w1, d0_g1, d0_b1, d0_w2, d0_og, d0_ob):
    n = N
    s0 = edge_index[0]
    d0 = edge_index[1]
    code = jnp.sort(jnp.concatenate([s0 * n + d0, d0 * n + s0]))
    su = code // n
    du = code % n
    first = jnp.concatenate([jnp.ones((1,), jnp.bool_), code[1:] != code[:-1]])

    rk = jax.random.key(42)
    rank = jnp.cumsum(first) - 1
    bits1 = _keep_bits(jax.random.fold_in(rk, 1), first)
    m1 = jax.random.permutation(jax.random.fold_in(rk, 2), n)[: n // 2]
    bits2 = _keep_bits(jax.random.fold_in(rk, 3), first)
    m2 = jax.random.permutation(jax.random.fold_in(rk, 4), n)[: n // 2]

    arangeE = jnp.arange(E2, dtype=jnp.int32)
    idx_v = jnp.where(first, rank, E2).astype(jnp.int32)
    tmp_f = jnp.zeros((E2,), jnp.float32).at[idx_v].add(
        (arangeE + 1).astype(jnp.float32), mode="drop", unique_indices=True)
    tmp = tmp_f.astype(jnp.int32) - 1
    b12 = bits1.astype(jnp.float32) + 2.0 * bits2.astype(jnp.float32)
    gb = jnp.zeros((E2,), jnp.float32).at[tmp].add(
        b12, mode="drop", unique_indices=True)
    bit2 = gb >= 2.0
    bit1 = (gb - 2.0 * bit2.astype(jnp.float32)) >= 1.0
    keep1 = first & bit1
    keep2 = first & bit2

    rm1 = jnp.ones((n,), jnp.float32).at[m1].set(0.0)
    rm2 = jnp.ones((n,), jnp.float32).at[m2].set(0.0)
    rm1c = rm1[:, None]
    rm2c = rm2[:, None]
    ones_c = jnp.ones((n, 1), jnp.float32)
    wm1c = 1.0 - rm1c
    wm2c = 1.0 - rm2c

    pad_i = (DUMMY + jnp.arange(E_CAP - E2, dtype=jnp.int32) % (AGG_ROWS - N))
    def interleave(a):
        return a.reshape(1000, E2 // 1000).T.reshape(-1)

    su_i = interleave(su)
    src3 = jnp.concatenate([su_i, jnp.zeros((E_CAP - E2,), jnp.int32)]).reshape(NW, KCH, CH)

    dummy_spread = DUMMY + (arangeE % (AGG_ROWS - N))

    def dst_arr(keep):
        dm = jnp.where(keep, du, dummy_spread).astype(jnp.int32)
        return jnp.concatenate([interleave(dm), pad_i]).reshape(NW, KCH, CH)

    dB1 = dst_arr(keep1)
    dB2 = dst_arr(keep2)

    zrows = jnp.zeros((ROWS_PER_TILE, D), jnp.float32)

    def gin(h, dst3, rmout, w1, g1, b1, w2, og, ob):
        partials = _spmm(src3, dst3, h, zrows)
        return _layer(h, partials, w1, g1[None, :], b1[None, :],
                      w2, og[None, :], ob[None, :], rmout)

    x1 = _rowmask(x, rm1c)
    h1 = gin(x1, dB1, ones_c, e0_w1, e0_g1, e0_b1, e0_w2, e0_og, e0_ob)
    h2 = gin(h1, dB1, rm1c, e1_w1, e1_g1, e1_b1, e1_w2, e1_og, e1_ob)
    r1 = gin(h2, dB1, ones_c, d0_w1, d0_g1, d0_b1, d0_w2, d0_og, d0_ob)

    x2 = _rowmask(x, rm2c)
    g1_ = gin(x2, dB2, ones_c, e0_w1, e0_g1, e0_b1, e0_w2, e0_og, e0_ob)
    g2 = gin(g1_, dB2, rm2c, e1_w1, e1_g1, e1_b1, e1_w2, e1_og, e1_ob)
    r2 = gin(g2, dB2, ones_c, d0_w1, d0_g1, d0_b1, d0_w2, d0_og, d0_ob)

    loss = _loss(x, r1, r2, wm1c, wm2c)
    return jnp.reshape(loss, ())


# speedup vs baseline: 1.0338x; 1.0338x over previous
import jax
import jax.numpy as jnp
from jax import lax
from jax.experimental import pallas as pl
from jax.experimental.pallas import tpu as pltpu
from jax.experimental.pallas import tpu_sc as plsc

N = 10000
D = 128
H = 512
E2 = 640000
DUMMY = N
AGG_ROWS = 10112
ROWS_PER_TILE = AGG_ROWS // 16
NW = 32
KCH = 160
CH = 128
E_CAP = NW * KCH * CH
RB = 1000
NRB = N // RB


GCH = 32
NGRP = KCH // GCH


def _spmm_body(src_hbm, dst_hbm, x_hbm, zrows_hbm, out_hbm,
               sidx, didx, buf0, buf1, sg0, sg1, agg_sh):
    c = lax.axis_index("c")
    s = lax.axis_index("s")
    wid = s * 2 + c
    tid = s
    pltpu.sync_copy(zrows_hbm, agg_sh.at[pl.ds(tid * ROWS_PER_TILE, ROWS_PER_TILE)])
    plsc.subcore_barrier()

    def g_start(ch, buf, sem):
        pltpu.make_async_copy(x_hbm.at[sidx.at[ch]], buf, sem).start()

    def g_wait(buf, sem):
        pltpu.make_async_copy(x_hbm.at[sidx.at[0]], buf, sem).wait()

    def s_add(ch, buf):
        pltpu.sync_copy(buf, agg_sh.at[didx.at[ch]], add=True)

    def group(grp, carry):
        pltpu.sync_copy(src_hbm.at[wid, pl.ds(grp * GCH, GCH)], sidx)
        pltpu.sync_copy(dst_hbm.at[wid, pl.ds(grp * GCH, GCH)], didx)
        g_start(0, buf0, sg0)
        g_start(1, buf1, sg1)

        def body(i, carry2):
            cc = i * 2
            g_wait(buf0, sg0)
            s_add(cc, buf0)
            g_start(cc + 2, buf0, sg0)
            g_wait(buf1, sg1)
            s_add(cc + 1, buf1)
            g_start(cc + 3, buf1, sg1)
            return carry2

        lax.fori_loop(0, GCH // 2 - 1, body, 0)
        g_wait(buf0, sg0)
        s_add(GCH - 2, buf0)
        g_wait(buf1, sg1)
        s_add(GCH - 1, buf1)
        return carry

    lax.fori_loop(0, NGRP, group, 0)
    plsc.subcore_barrier()
    pltpu.sync_copy(agg_sh.at[pl.ds(tid * ROWS_PER_TILE, ROWS_PER_TILE)],
                    out_hbm.at[c, pl.ds(tid * ROWS_PER_TILE, ROWS_PER_TILE)])


import functools


@functools.cache
def _spmm_kernel():
    return pl.kernel(
        _spmm_body,
        out_type=jax.ShapeDtypeStruct((2, AGG_ROWS, D), jnp.float32),
        mesh=plsc.VectorSubcoreMesh(core_axis_name="c", subcore_axis_name="s",
                                    num_cores=2, num_subcores=16),
        scratch_types=[
            pltpu.VMEM((GCH, CH), jnp.int32),
            pltpu.VMEM((GCH, CH), jnp.int32),
            pltpu.VMEM((CH, D), jnp.float32),
            pltpu.VMEM((CH, D), jnp.float32),
            pltpu.SemaphoreType.DMA,
            pltpu.SemaphoreType.DMA,
            pltpu.VMEM_SHARED((AGG_ROWS, D), jnp.float32),
        ],
    )


def _spmm(src3, dst3, h, zrows):
    return _spmm_kernel()(src3, dst3, h, zrows)


def _layer_body(x_ref, a_ref, w1_ref, g1_ref, b1_ref, w2_ref,
                og_ref, ob_ref, rmout_ref, o_ref, t_scr, y_scr, s1_scr, s2_scr):
    p = pl.program_id(0)
    b = pl.program_id(1)

    @pl.when(p == 0)
    def _():
        h = x_ref[...] + a_ref[0] + a_ref[1]
        t = jnp.dot(h, w1_ref[...], preferred_element_type=jnp.float32)
        t_scr[pl.ds(b * RB, RB), :] = t

        @pl.when(b == 0)
        def _():
            s1_scr[...] = jnp.zeros_like(s1_scr)

        s1_scr[0:1, :] += jnp.sum(t, axis=0, keepdims=True)
        s1_scr[1:2, :] += jnp.sum(t * t, axis=0, keepdims=True)

    @pl.when(p == 1)
    def _():
        t = t_scr[pl.ds(b * RB, RB), :]
        m = s1_scr[0:1, :] * (1.0 / N)
        v = s1_scr[1:2, :] * (1.0 / N) - m * m
        tn = g1_ref[...] * (t - m) / jnp.sqrt(v + 1e-5) + b1_ref[...]
        tr = jnp.maximum(tn, 0.0)
        y = jnp.dot(tr, w2_ref[...], preferred_element_type=jnp.float32)
        y_scr[pl.ds(b * RB, RB), :] = y

        @pl.when(b == 0)
        def _():
            s2_scr[...] = jnp.zeros_like(s2_scr)

        s2_scr[0:1, :] += jnp.sum(y, axis=0, keepdims=True)
        s2_scr[1:2, :] += jnp.sum(y * y, axis=0, keepdims=True)

    @pl.when(p == 2)
    def _():
        y = y_scr[pl.ds(b * RB, RB), :]
        m = s2_scr[0:1, :] * (1.0 / N)
        v = s2_scr[1:2, :] * (1.0 / N) - m * m
        yn = og_ref[...] * (y - m) / jnp.sqrt(v + 1e-5) + ob_ref[...]
        o_ref[...] = jnp.maximum(yn, 0.0) * rmout_ref[...]


def _layer(h, partials, w1, g1, b1, w2, og, ob, rmout):
    return pl.pallas_call(
        _layer_body,
        grid=(3, NRB),
        in_specs=[
            pl.BlockSpec((RB, D), lambda p, b: ((p == 0) * b, 0)),
            pl.BlockSpec((2, RB, D), lambda p, b: (0, (p == 0) * b, 0)),
            pl.BlockSpec((D, H), lambda p, b: (0, 0)),
            pl.BlockSpec((1, H), lambda p, b: (0, 0)),
            pl.BlockSpec((1, H), lambda p, b: (0, 0)),
            pl.BlockSpec((H, D), lambda p, b: (0, 0)),
            pl.BlockSpec((1, D), lambda p, b: (0, 0)),
            pl.BlockSpec((1, D), lambda p, b: (0, 0)),
            pl.BlockSpec((RB, 1), lambda p, b: ((p == 2) * b, 0)),
        ],
        out_specs=pl.BlockSpec((RB, D), lambda p, b: ((p == 2) * b, 0)),
        out_shape=jax.ShapeDtypeStruct((N, D), jnp.float32),
        scratch_shapes=[
            pltpu.VMEM((N, H), jnp.float32),
            pltpu.VMEM((N, D), jnp.float32),
            pltpu.VMEM((2, H), jnp.float32),
            pltpu.VMEM((2, D), jnp.float32),
        ],
    )(h, partials, w1, g1, b1, w2, og, ob, rmout)


def _rowmask_body(x_ref, rm_ref, o_ref):
    o_ref[...] = x_ref[...] * rm_ref[...]


def _rowmask(x, rm):
    return pl.pallas_call(
        _rowmask_body,
        grid=(NRB,),
        in_specs=[
            pl.BlockSpec((RB, D), lambda b: (b, 0)),
            pl.BlockSpec((RB, 1), lambda b: (b, 0)),
        ],
        out_specs=pl.BlockSpec((RB, D), lambda b: (b, 0)),
        out_shape=jax.ShapeDtypeStruct((N, D), jnp.float32),
    )(x, rm)


def _loss_body(x_ref, r1_ref, r2_ref, wm1_ref, wm2_ref, o_ref):
    def nrm(a):
        nn = jnp.sqrt(jnp.sum(a * a, axis=1, keepdims=True))
        return a / jnp.maximum(nn, 1e-12)

    xn = nrm(x_ref[...])
    r1n = nrm(r1_ref[...])
    r2n = nrm(r2_ref[...])
    c1 = jnp.sum(r1n * xn, axis=1, keepdims=True)
    c2 = jnp.sum(r2n * xn, axis=1, keepdims=True)
    ccl = jnp.sum(r2n * r1n, axis=1, keepdims=True)
    l1 = jnp.sum(wm1_ref[...] * (1.0 - c1)) / (N // 2)
    l2 = jnp.sum(wm2_ref[...] * (1.0 - c2)) / (N // 2)
    cl = jnp.sum(1.0 - ccl) / N
    o_ref[...] = jnp.reshape(l1 + l2 + 0.1 * cl, (1, 1))


def _loss(x, r1, r2, wm1, wm2):
    return pl.pallas_call(
        _loss_body,
        out_shape=jax.ShapeDtypeStruct((1, 1), jnp.float32),
    )(x, r1, r2, wm1, wm2)


def _tf2x32(k0, k1, x0, x1):
    ks = (k0, k1, k0 ^ k1 ^ jnp.uint32(0x1BD11BDA))
    rotations = ((13, 15, 26, 6), (17, 29, 16, 24))
    x0 = x0 + ks[0]
    x1 = x1 + ks[1]
    for i in range(5):
        for r in rotations[i % 2]:
            x0 = x0 + x1
            x1 = (x1 << r) | (x1 >> (32 - r))
            x1 = x0 ^ x1
        x0 = x0 + ks[(i + 1) % 3]
        x1 = x1 + ks[(i + 2) % 3] + jnp.uint32(i + 1)
    return x0, x1


def _unif_prefix(key, count, size):
    kd = jax.random.key_data(key)
    c = count.astype(jnp.uint32)
    half = (c + jnp.uint32(1)) // jnp.uint32(2)
    odd = c & jnp.uint32(1)
    j = jnp.arange(size, dtype=jnp.uint32)
    second = j >= half
    a0 = jnp.where(second, j - half, j)
    pad = (odd == jnp.uint32(1)) & (j == half - jnp.uint32(1))
    a1 = jnp.where(second, j, jnp.where(pad, jnp.uint32(0), j + half))
    o0, o1 = _tf2x32(kd[0], kd[1], a0, a1)
    bits = jnp.where(second, o1, o0)
    f = jax.lax.bitcast_convert_type((bits >> 9) | jnp.uint32(0x3F800000), jnp.float32)
    return f - jnp.float32(1.0)


def _keep_bits(key, valid, p=0.2):
    size = valid.shape[0]
    if jax.config.jax_threefry_partitionable:
        return jax.random.bernoulli(key, 1.0 - p, (size,))
    u = _unif_prefix(key, valid.sum(), size)
    return u < jnp.float32(1.0 - p)


def kernel(x, edge_index, batch, e0_w1, e0_g1, e0_b1, e0_w2, e0_og, e0_ob,
           e1_w1, e1_g1, e1_b1, e1_w2, e1_og, e1_ob,
           d0_w1, d0_g1, d0_b1, d0_w2, d0_og, d0_ob):
    n = N
    s0 = edge_index[0]
    d0 = edge_index[1]
    code = jnp.sort(jnp.concatenate([s0 * n + d0, d0 * n + s0]))
    su = code // n
    du = code % n
    first = jnp.concatenate([jnp.ones((1,), jnp.bool_), code[1:] != code[:-1]])

    rk = jax.random.key(42)
    rank = jnp.cumsum(first) - 1
    bits1 = _keep_bits(jax.random.fold_in(rk, 1), first)
    m1 = jax.random.permutation(jax.random.fold_in(rk, 2), n)[: n // 2]
    bits2 = _keep_bits(jax.random.fold_in(rk, 3), first)
    m2 = jax.random.permutation(jax.random.fold_in(rk, 4), n)[: n // 2]

    arangeE = jnp.arange(E2, dtype=jnp.int32)
    idx_v = jnp.where(first, rank, E2).astype(jnp.int32)
    tmp_f = jnp.zeros((E2,), jnp.float32).at[idx_v].add(
        (arangeE + 1).astype(jnp.float32), mode="drop", unique_indices=True)
    tmp = tmp_f.astype(jnp.int32) - 1
    b12 = bits1.astype(jnp.float32) + 2.0 * bits2.astype(jnp.float32)
    gb = jnp.zeros((E2,), jnp.float32).at[tmp].add(
        b12, mode="drop", unique_indices=True)
    bit2 = gb >= 2.0
    bit1 = (gb - 2.0 * bit2.astype(jnp.float32)) >= 1.0
    keep1 = first & bit1
    keep2 = first & bit2

    rm1 = jnp.ones((n,), jnp.float32).at[m1].set(0.0)
    rm2 = jnp.ones((n,), jnp.float32).at[m2].set(0.0)
    rm1c = rm1[:, None]
    rm2c = rm2[:, None]
    ones_c = jnp.ones((n, 1), jnp.float32)
    wm1c = 1.0 - rm1c
    wm2c = 1.0 - rm2c

    pad_i = (DUMMY + jnp.arange(E_CAP - E2, dtype=jnp.int32) % (AGG_ROWS - N))
    def interleave(a):
        return a.reshape(1000, E2 // 1000).T.reshape(-1)

    su_i = interleave(su)
    src3 = jnp.concatenate([su_i, jnp.zeros((E_CAP - E2,), jnp.int32)]).reshape(NW, KCH, CH)

    dummy_spread = DUMMY + (arangeE % (AGG_ROWS - N))

    def dst_arr(keep):
        dm = jnp.where(keep, du, dummy_spread).astype(jnp.int32)
        return jnp.concatenate([interleave(dm), pad_i]).reshape(NW, KCH, CH)

    dB1 = dst_arr(keep1)
    dB2 = dst_arr(keep2)

    zrows = jnp.zeros((ROWS_PER_TILE, D), jnp.float32)

    def gin(h, dst3, rmout, w1, g1, b1, w2, og, ob):
        partials = _spmm(src3, dst3, h, zrows)
        return _layer(h, partials, w1, g1[None, :], b1[None, :],
                      w2, og[None, :], ob[None, :], rmout)

    x1 = _rowmask(x, rm1c)
    h1 = gin(x1, dB1, ones_c, e0_w1, e0_g1, e0_b1, e0_w2, e0_og, e0_ob)
    h2 = gin(h1, dB1, rm1c, e1_w1, e1_g1, e1_b1, e1_w2, e1_og, e1_ob)
    r1 = gin(h2, dB1, ones_c, d0_w1, d0_g1, d0_b1, d0_w2, d0_og, d0_ob)

    x2 = _rowmask(x, rm2c)
    g1_ = gin(x2, dB2, ones_c, e0_w1, e0_g1, e0_b1, e0_w2, e0_og, e0_ob)
    g2 = gin(g1_, dB2, rm2c, e1_w1, e1_g1, e1_b1, e1_w2, e1_og, e1_ob)
    r2 = gin(g2, dB2, ones_c, d0_w1, d0_g1, d0_b1, d0_w2, d0_og, d0_ob)

    loss = _loss(x, r1, r2, wm1c, wm2c)
    return jnp.reshape(loss, ())
